# Initial kernel scaffold; baseline (speedup 1.0000x reference)
import jax
import jax.numpy as jnp
from jax.experimental import pallas as pl


def kernel(x, W_router, compress_neurons):
    raise NotImplementedError("write your pallas kernel here")


# speedup vs baseline: 6.0528x; 6.0528x over previous
import functools

import jax
import jax.numpy as jnp
from jax import lax
from jax.experimental import pallas as pl

B, S, D_MODEL = 1, 2048, 768
RANK = 32
N_COMPRESS = 64
TOP_K = 2

BLK = 256
NEG = -1e30


def _tc_body(x_ref, wr_ref, wf_ref, out_ref, w_ref, idx_ref):
    x_blk = x_ref[...]
    scores = jnp.dot(x_blk, wr_ref[...], preferred_element_type=jnp.float32)
    iota_n = lax.broadcasted_iota(jnp.int32, (BLK, N_COMPRESS), 1)
    m1 = jnp.max(scores, axis=1, keepdims=True)
    i1 = jnp.min(jnp.where(scores == m1, iota_n, N_COMPRESS), axis=1,
                 keepdims=True)
    masked = jnp.where(iota_n == i1, NEG, scores)
    m2 = jnp.max(masked, axis=1, keepdims=True)
    i2 = jnp.min(jnp.where(masked == m2, iota_n, N_COMPRESS), axis=1,
                 keepdims=True)
    e = jnp.exp(m2 - m1)
    w1 = 1.0 / (1.0 + e)
    w2 = 1.0 - w1
    w_ref[...] = jnp.concatenate([w1, w2], axis=1)
    idx_ref[...] = jnp.concatenate([i1, i2], axis=1)

    proj = jnp.dot(x_blk, wf_ref[...], preferred_element_type=jnp.float32)
    col_n = lax.broadcasted_iota(jnp.int32, (BLK, N_COMPRESS * RANK), 1) // RANK
    mask = (w1 * (col_n == i1).astype(jnp.float32)
            + w2 * (col_n == i2).astype(jnp.float32))
    row = lax.broadcasted_iota(jnp.int32, (N_COMPRESS * RANK, RANK), 0) % RANK
    col = lax.broadcasted_iota(jnp.int32, (N_COMPRESS * RANK, RANK), 1)
    gather_eye = (row == col).astype(jnp.float32)
    out_ref[...] = jnp.dot(proj * mask, gather_eye,
                           preferred_element_type=jnp.float32)


@jax.jit
def kernel(x, W_router, compress_neurons):
    x2d = x.reshape(S, D_MODEL)
    wr_t = W_router.T
    wf = compress_neurons.transpose(1, 0, 2).reshape(D_MODEL,
                                                     N_COMPRESS * RANK)
    grid = (S // BLK,)
    out, w, idx = pl.pallas_call(
        _tc_body,
        grid=grid,
        in_specs=[
            pl.BlockSpec((BLK, D_MODEL), lambda i: (i, 0)),
            pl.BlockSpec((D_MODEL, N_COMPRESS), lambda i: (0, 0)),
            pl.BlockSpec((D_MODEL, N_COMPRESS * RANK), lambda i: (0, 0)),
        ],
        out_specs=[
            pl.BlockSpec((BLK, RANK), lambda i: (i, 0)),
            pl.BlockSpec((BLK, TOP_K), lambda i: (i, 0)),
            pl.BlockSpec((BLK, TOP_K), lambda i: (i, 0)),
        ],
        out_shape=[
            jax.ShapeDtypeStruct((S, RANK), jnp.float32),
            jax.ShapeDtypeStruct((S, TOP_K), jnp.float32),
            jax.ShapeDtypeStruct((S, TOP_K), jnp.int32),
        ],
    )(x2d, wr_t, wf)
    return (out.reshape(B, S, RANK), w.reshape(B, S, TOP_K),
            idx.reshape(B, S, TOP_K))
